# baseline (speedup 1.0000x reference)
import functools

import jax
import jax.numpy as jnp
import numpy as np
from jax import lax
from jax.experimental import pallas as pl
from jax.experimental.pallas import tpu as pltpu
from jax.experimental.pallas import tpu_sc as plsc

KNN = 16
CUTOFF = 5.0
PB = 256


def _knn_body(pos_ref, posT_ref, nbr_ref, rbf_ref, *, P, num_rbf):
    b = pl.program_id(0)
    i = pl.program_id(1)
    xd = pos_ref[0]
    xs = posT_ref[0]
    xd0, xd1, xd2 = xd[:, 0:1], xd[:, 1:2], xd[:, 2:3]
    xs0, xs1, xs2 = xs[0:1, :], xs[1:2, :], xs[2:3, :]
    d2 = (xd0 - xs0) ** 2 + (xd1 - xs1) ** 2 + (xd2 - xs2) ** 2
    col = lax.broadcasted_iota(jnp.int32, (PB, P), 1)
    rowg = i * PB + lax.broadcasted_iota(jnp.int32, (PB, P), 0)
    d2 = jnp.where(col == rowg, 1e9, d2)
    col_f = col.astype(jnp.float32)

    centers = lax.broadcasted_iota(
        jnp.int32, (1, num_rbf), 1).astype(jnp.float32) * jnp.float32(
            CUTOFF / (num_rbf - 1))
    gamma = jnp.float32(num_rbf / CUTOFF)

    rs = []
    for k in range(KNN):
        m = jnp.min(d2, axis=1, keepdims=True)
        idxf = jnp.min(jnp.where(d2 == m, col_f, jnp.float32(P)),
                       axis=1, keepdims=True)
        onehot = col_f == idxf
        rs.append(jnp.sqrt(m + 1e-12))
        nbr_ref[0, :, k:k + 1] = idxf.astype(jnp.int32) + b * P
        d2 = jnp.where(onehot, 1e9, d2)

    r_all = jnp.concatenate(rs, axis=1)
    env_all = 0.5 * (jnp.cos(jnp.float32(np.pi) *
                             jnp.clip(r_all / CUTOFF, 0.0, 1.0)) + 1.0)
    for k in range(KNN):
        rbf_k = jnp.exp(-gamma * (rs[k] - centers) ** 2) * env_all[:, k:k + 1]
        rbf_ref[0, k] = rbf_k


def _edge_body(psrc_ref, pos_ref, rhb_ref, *, C):
    ps = psrc_ref[0]
    xd = pos_ref[0]
    rel = ps[..., 0:3] - xd[None, :, :]
    r = jnp.sqrt(rel[..., 0:1] ** 2 + rel[..., 1:2] ** 2 +
                 rel[..., 2:3] ** 2 + 1e-12)
    rh = rel / r
    rhb_ref[0] = jnp.concatenate(
        [jnp.broadcast_to(rh[..., 0:1], (KNN, PB, C)),
         jnp.broadcast_to(rh[..., 1:2], (KNN, PB, C)),
         jnp.broadcast_to(rh[..., 2:3], (KNN, PB, C))], axis=2)


def _layer_body(g_ref, rbf_ref, rhb_ref, tbl_ref,
                W1_ref, b1_ref, W2_ref, b2_ref,
                Wg_ref, bg_ref, Wcat_ref, out_ref, *, C):
    rb2 = rbf_ref[0].reshape(KNN * PB, rbf_ref.shape[-1])
    h = rb2 @ W1_ref[...] + b1_ref[...]
    h = h * jax.nn.sigmoid(h)
    w2d = h @ W2_ref[...] + b2_ref[...]
    w = w2d.reshape(KNN, PB, 8 * C)
    wss = w[..., 0:C]
    wvs = w[..., C:2 * C]
    wsv3 = w[..., 2 * C:5 * C]
    wvv3 = w[..., 5 * C:8 * C]
    G = g_ref[0]
    s = G[..., 0:C]
    V = G[..., C:4 * C]
    RB = rhb_ref[0]
    VR = V * RB
    vdot = VR[..., 0:C] + VR[..., C:2 * C] + VR[..., 2 * C:3 * C]
    msg_s = wss * s + wvs * vdot
    sr = wsv3 * RB
    msg_v = wvv3 * V + sr * jnp.concatenate([s, s, s], axis=2)
    inv_k = jnp.float32(1.0 / KNN)
    agg_s = jnp.sum(msg_s, axis=0) * inv_k
    agg_v = jnp.sum(msg_v, axis=0) * inv_k
    gate = jax.nn.sigmoid(agg_s @ Wg_ref[...] + bg_ref[...])
    lhs = jnp.concatenate(
        [agg_s * jax.nn.sigmoid(agg_s),
         gate * agg_v[:, 0:C],
         gate * agg_v[:, C:2 * C],
         gate * agg_v[:, 2 * C:3 * C]], axis=0)
    mm = lhs @ Wcat_ref[...]
    upd = jnp.concatenate(
        [mm[0:PB, 0:C], mm[PB:2 * PB, C:2 * C],
         mm[2 * PB:3 * PB, C:2 * C], mm[3 * PB:4 * PB, C:2 * C]], axis=1)
    out_ref[0] = tbl_ref[0] + upd


def _layer0_body(rbf_ref, rhb_ref, emb_ref,
                 W1_ref, b1_ref, W2_ref, b2_ref,
                 Wg_ref, bg_ref, Wcat_ref, out_ref, *, C):
    rb2 = rbf_ref[0].reshape(KNN * PB, rbf_ref.shape[-1])
    h = rb2 @ W1_ref[...] + b1_ref[...]
    h = h * jax.nn.sigmoid(h)
    w2d = h @ W2_ref[...] + b2_ref[...]
    w = w2d.reshape(KNN, PB, 8 * C)
    wss = w[..., 0:C]
    wsv3 = w[..., 2 * C:5 * C]
    RB = rhb_ref[0]
    emb = emb_ref[...]
    msg_s = wss * emb[0:1, None, 0:C]
    msg_v = (wsv3 * RB) * emb[0:1, None, C:4 * C]
    inv_k = jnp.float32(1.0 / KNN)
    agg_s = jnp.sum(msg_s, axis=0) * inv_k
    agg_v = jnp.sum(msg_v, axis=0) * inv_k
    gate = jax.nn.sigmoid(agg_s @ Wg_ref[...] + bg_ref[...])
    lhs = jnp.concatenate(
        [agg_s * jax.nn.sigmoid(agg_s),
         gate * agg_v[:, 0:C],
         gate * agg_v[:, C:2 * C],
         gate * agg_v[:, 2 * C:3 * C]], axis=0)
    mm = lhs @ Wcat_ref[...]
    upd = jnp.concatenate(
        [mm[0:PB, 0:C] + emb[0:1, 0:C], mm[PB:2 * PB, C:2 * C],
         mm[2 * PB:3 * PB, C:2 * C], mm[3 * PB:4 * PB, C:2 * C]], axis=1)
    out_ref[0] = upd


def _readout_body(tbl_ref, Wc1_ref, bc1_ref, Wc2_ref, bc2_ref,
                  Wc3_ref, bc3_ref, out_ref, *, C):
    s = tbl_ref[0][:, 0:C]
    pooled = jnp.mean(s, axis=0, keepdims=True)
    h = pooled @ Wc1_ref[...] + bc1_ref[...]
    h = h * jax.nn.sigmoid(h)
    h = h @ Wc2_ref[...] + bc2_ref[...]
    h = h * jax.nn.sigmoid(h)
    out_ref[0] = h @ Wc3_ref[...] + bc3_ref[...]


def _make_sc_gather(R, D, n_workers, chunk, tc_tiling=True):
    per_w = R // n_workers
    n_chunks = per_w // chunk
    mesh = plsc.VectorSubcoreMesh(core_axis_name="c", subcore_axis_name="s")
    info = plsc.get_sparse_core_info()
    nc = info.num_cores

    @functools.partial(
        pl.kernel, mesh=mesh,
        out_type=jax.ShapeDtypeStruct((R, D), jnp.float32),
        compiler_params=pltpu.CompilerParams(use_tc_tiling_on_sc=tc_tiling),
        scratch_types=[
            pltpu.VMEM((chunk,), jnp.int32),
            pltpu.VMEM((chunk,), jnp.int32),
            pltpu.VMEM((chunk, D), jnp.float32),
            pltpu.VMEM((chunk, D), jnp.float32),
            pltpu.SemaphoreType.DMA,
            pltpu.SemaphoreType.DMA,
            pltpu.SemaphoreType.DMA,
            pltpu.SemaphoreType.DMA,
        ],
    )
    def gather_k(table_hbm, idx_hbm, out_hbm, idx_v0, idx_v1,
                 rows_v0, rows_v1, g0, g1, s0, s1):
        assert n_chunks % 2 == 0 and n_chunks >= 2
        wid = lax.axis_index("s") * nc + lax.axis_index("c")
        idx_v = (idx_v0, idx_v1)
        rows_v = (rows_v0, rows_v1)
        gsem = (g0, g1)
        ssem = (s0, s1)

        def start(j, slot):
            base = pl.multiple_of(wid * per_w + j * chunk, 8)
            pltpu.sync_copy(idx_hbm.at[pl.ds(base, chunk)], idx_v[slot])
            pltpu.async_copy(table_hbm.at[idx_v[slot]], rows_v[slot],
                             gsem[slot])

        def wait_scatter(j, slot):
            base = pl.multiple_of(wid * per_w + j * chunk, 8)
            pltpu.make_async_copy(rows_v[slot], out_hbm.at[pl.ds(base, chunk)],
                                  ssem[slot]).wait()

        start(0, 0)

        def body(jj, carry):
            for b in range(2):
                j = 2 * jj + b
                nb = 1 - b

                @pl.when(j + 1 < n_chunks)
                def _():
                    @pl.when(j >= 1)
                    def _():
                        wait_scatter(j - 1, nb)
                    start(j + 1, nb)

                base = pl.multiple_of(wid * per_w + j * chunk, 8)
                pltpu.make_async_copy(table_hbm.at[idx_v[b]], rows_v[b],
                                      gsem[b]).wait()
                pltpu.async_copy(rows_v[b], out_hbm.at[pl.ds(base, chunk)],
                                 ssem[b])
            return carry

        lax.fori_loop(0, n_chunks // 2, body, 0)
        wait_scatter(n_chunks - 2, 0)
        wait_scatter(n_chunks - 1, 1)

    return gather_k


def kernel(batch, embed_w, W1, b1, W2, b2, Wg, bg, Wms, Wmv,
           Wc1, bc1, Wc2, bc2, Wc3, bc3):
    B, P, _ = batch.shape
    C = embed_w.shape[1]
    num_rbf = W1.shape[1]
    RH = W1.shape[2]
    L = W1.shape[0]
    ncls = Wc3.shape[1]
    nblk = P // PB

    batchT = jnp.transpose(batch, (0, 2, 1))

    nbr, rbf = pl.pallas_call(
        functools.partial(_knn_body, P=P, num_rbf=num_rbf),
        grid=(B, nblk),
        in_specs=[
            pl.BlockSpec((1, PB, 3), lambda b, i: (b, i, 0)),
            pl.BlockSpec((1, 3, P), lambda b, i: (b, 0, 0)),
        ],
        out_specs=[
            pl.BlockSpec((1, PB, KNN), lambda b, i: (b, i, 0)),
            pl.BlockSpec((1, KNN, PB, num_rbf), lambda b, i: (b, 0, i, 0)),
        ],
        out_shape=[
            jax.ShapeDtypeStruct((B, P, KNN), jnp.int32),
            jax.ShapeDtypeStruct((B, KNN, P, num_rbf), jnp.float32),
        ],
    )(batch, batchT)

    idx = jnp.transpose(nbr, (0, 2, 1)).reshape(B * P * KNN)
    R = B * P * KNN
    D = 4 * C
    sc_gather_h = _make_sc_gather(R // 2, D, 32, 256)
    sc_gather_pos = _make_sc_gather(R, 16, 32, 512, tc_tiling=False)

    pos_table = jnp.pad(batch.reshape(B * P, 3), ((0, 0), (0, 13)))
    psrc = sc_gather_pos(pos_table, idx)

    rhb = pl.pallas_call(
        functools.partial(_edge_body, C=C),
        grid=(B, nblk),
        in_specs=[
            pl.BlockSpec((1, KNN, PB, 16), lambda b, i: (b, 0, i, 0)),
            pl.BlockSpec((1, PB, 3), lambda b, i: (b, i, 0)),
        ],
        out_specs=pl.BlockSpec((1, KNN, PB, 3 * C), lambda b, i: (b, 0, i, 0)),
        out_shape=jax.ShapeDtypeStruct((B, KNN, P, 3 * C), jnp.float32),
    )(psrc.reshape(B, KNN, P, 16), batch)

    emb4 = jnp.concatenate([embed_w, embed_w, embed_w, embed_w], axis=1)

    full = lambda shape: pl.BlockSpec(shape, lambda b, i: tuple(0 for _ in shape))
    def make_layer_call(nb):
        return pl.pallas_call(
            functools.partial(_layer_body, C=C),
            grid=(nb, nblk),
            in_specs=[
                pl.BlockSpec((1, KNN, PB, 4 * C), lambda b, i: (b, 0, i, 0)),
                pl.BlockSpec((1, KNN, PB, num_rbf), lambda b, i: (b, 0, i, 0)),
                pl.BlockSpec((1, KNN, PB, 3 * C), lambda b, i: (b, 0, i, 0)),
                pl.BlockSpec((1, PB, 4 * C), lambda b, i: (b, i, 0)),
                full((num_rbf, RH)), full((1, RH)),
                full((RH, 8 * C)), full((1, 8 * C)),
                full((C, C)), full((1, C)), full((C, 2 * C)),
            ],
            out_specs=pl.BlockSpec((1, PB, 4 * C), lambda b, i: (b, i, 0)),
            out_shape=jax.ShapeDtypeStruct((nb, P, 4 * C), jnp.float32),
        )

    layer_call_h = make_layer_call(B // 2)

    layer0_call = pl.pallas_call(
        functools.partial(_layer0_body, C=C),
        grid=(B, nblk),
        in_specs=[
            pl.BlockSpec((1, KNN, PB, num_rbf), lambda b, i: (b, 0, i, 0)),
            pl.BlockSpec((1, KNN, PB, 3 * C), lambda b, i: (b, 0, i, 0)),
            full((1, 4 * C)),
            full((num_rbf, RH)), full((1, RH)),
            full((RH, 8 * C)), full((1, 8 * C)),
            full((C, C)), full((1, C)), full((C, 2 * C)),
        ],
        out_specs=pl.BlockSpec((1, PB, 4 * C), lambda b, i: (b, i, 0)),
        out_shape=jax.ShapeDtypeStruct((B, P, 4 * C), jnp.float32),
    )

    table = None
    for l in range(L):
        W2l, b2l = W2[l], b2[l]
        W2rep = jnp.concatenate(
            [W2l[:, 0:2 * C],
             W2l[:, 2 * C:3 * C], W2l[:, 2 * C:3 * C], W2l[:, 2 * C:3 * C],
             W2l[:, 3 * C:4 * C], W2l[:, 3 * C:4 * C], W2l[:, 3 * C:4 * C]],
            axis=1)
        b2rep = jnp.concatenate(
            [b2l[0:2 * C],
             b2l[2 * C:3 * C], b2l[2 * C:3 * C], b2l[2 * C:3 * C],
             b2l[3 * C:4 * C], b2l[3 * C:4 * C], b2l[3 * C:4 * C]])
        Wcat = jnp.concatenate([Wms[l], Wmv[l]], axis=1)
        if l == 0:
            table3 = layer0_call(
                rbf, rhb, emb4,
                W1[l], b1[l][None, :], W2rep, b2rep[None, :],
                Wg[l], bg[l][None, :], Wcat)
        else:
            hb = B // 2
            tbl3 = table.reshape(B, P, 4 * C)
            g1 = sc_gather_h(table, idx[:R // 2])
            g2 = sc_gather_h(table, idx[R // 2:])
            halves = []
            for hi, gh in enumerate((g1, g2)):
                halves.append(layer_call_h(
                    gh.reshape(hb, KNN, P, 4 * C),
                    rbf[hi * hb:(hi + 1) * hb], rhb[hi * hb:(hi + 1) * hb],
                    tbl3[hi * hb:(hi + 1) * hb],
                    W1[l], b1[l][None, :], W2rep, b2rep[None, :],
                    Wg[l], bg[l][None, :], Wcat))
            table3 = jnp.concatenate(halves, axis=0)
        table = table3.reshape(B * P, 4 * C)

    out = pl.pallas_call(
        functools.partial(_readout_body, C=C),
        grid=(B,),
        in_specs=[
            pl.BlockSpec((1, P, 4 * C), lambda b: (b, 0, 0)),
            pl.BlockSpec((C, 128), lambda b: (0, 0)),
            pl.BlockSpec((1, 128), lambda b: (0, 0)),
            pl.BlockSpec((128, 64), lambda b: (0, 0)),
            pl.BlockSpec((1, 64), lambda b: (0, 0)),
            pl.BlockSpec((64, ncls), lambda b: (0, 0)),
            pl.BlockSpec((1, ncls), lambda b: (0, 0)),
        ],
        out_specs=pl.BlockSpec((1, 1, ncls), lambda b: (b, 0, 0)),
        out_shape=jax.ShapeDtypeStruct((B, 1, ncls), jnp.float32),
    )(table.reshape(B, P, 4 * C), Wc1, bc1[None, :], Wc2, bc2[None, :],
      Wc3, bc3[None, :])

    return out.reshape(B, ncls)


# speedup vs baseline: 1.0473x; 1.0473x over previous
import functools

import jax
import jax.numpy as jnp
import numpy as np
from jax import lax
from jax.experimental import pallas as pl
from jax.experimental.pallas import tpu as pltpu
from jax.experimental.pallas import tpu_sc as plsc

KNN = 16
CUTOFF = 5.0
PB = 256


def _knn_body(pos_ref, posT_ref, nbr_ref, rbf_ref, *, P, num_rbf):
    b = pl.program_id(0)
    i = pl.program_id(1)
    xd = pos_ref[0]
    xs = posT_ref[0]
    xd0, xd1, xd2 = xd[:, 0:1], xd[:, 1:2], xd[:, 2:3]
    xs0, xs1, xs2 = xs[0:1, :], xs[1:2, :], xs[2:3, :]
    d2 = (xd0 - xs0) ** 2 + (xd1 - xs1) ** 2 + (xd2 - xs2) ** 2
    col = lax.broadcasted_iota(jnp.int32, (PB, P), 1)
    rowg = i * PB + lax.broadcasted_iota(jnp.int32, (PB, P), 0)
    d2 = jnp.where(col == rowg, 1e9, d2)
    col_f = col.astype(jnp.float32)

    centers = lax.broadcasted_iota(
        jnp.int32, (1, num_rbf), 1).astype(jnp.float32) * jnp.float32(
            CUTOFF / (num_rbf - 1))
    gamma = jnp.float32(num_rbf / CUTOFF)

    rs = []
    for k in range(KNN):
        m = jnp.min(d2, axis=1, keepdims=True)
        idxf = jnp.min(jnp.where(d2 == m, col_f, jnp.float32(P)),
                       axis=1, keepdims=True)
        onehot = col_f == idxf
        rs.append(jnp.sqrt(m + 1e-12))
        nbr_ref[0, :, k:k + 1] = idxf.astype(jnp.int32) + b * P
        d2 = jnp.where(onehot, 1e9, d2)

    r_all = jnp.concatenate(rs, axis=1)
    env_all = 0.5 * (jnp.cos(jnp.float32(np.pi) *
                             jnp.clip(r_all / CUTOFF, 0.0, 1.0)) + 1.0)
    for k in range(KNN):
        rbf_k = jnp.exp(-gamma * (rs[k] - centers) ** 2) * env_all[:, k:k + 1]
        rbf_ref[0, k] = rbf_k


def _edge_body(psrc_ref, pos_ref, rhb_ref, *, C):
    ps = psrc_ref[0]
    xd = pos_ref[0]
    rel = ps[..., 0:3] - xd[None, :, :]
    r = jnp.sqrt(rel[..., 0:1] ** 2 + rel[..., 1:2] ** 2 +
                 rel[..., 2:3] ** 2 + 1e-12)
    rh = rel / r
    rhb_ref[0] = jnp.concatenate(
        [jnp.broadcast_to(rh[..., 0:1], (KNN, PB, C)),
         jnp.broadcast_to(rh[..., 1:2], (KNN, PB, C)),
         jnp.broadcast_to(rh[..., 2:3], (KNN, PB, C))], axis=2)


def _layer_body(g_ref, rbf_ref, rhb_ref, tbl_ref,
                W1_ref, b1_ref, W2_ref, b2_ref,
                Wg_ref, bg_ref, Wcat_ref, out_ref, *, C):
    rb2 = rbf_ref[0].reshape(KNN * PB, rbf_ref.shape[-1])
    h = rb2 @ W1_ref[...] + b1_ref[...]
    h = h * jax.nn.sigmoid(h)
    w2d = h @ W2_ref[...] + b2_ref[...]
    w = w2d.reshape(KNN, PB, 8 * C)
    wss = w[..., 0:C]
    wvs = w[..., C:2 * C]
    wsv3 = w[..., 2 * C:5 * C]
    wvv3 = w[..., 5 * C:8 * C]
    G = g_ref[0]
    s = G[..., 0:C]
    V = G[..., C:4 * C]
    RB = rhb_ref[0]
    VR = V * RB
    vdot = VR[..., 0:C] + VR[..., C:2 * C] + VR[..., 2 * C:3 * C]
    msg_s = wss * s + wvs * vdot
    sr = wsv3 * RB
    msg_v = wvv3 * V + sr * jnp.concatenate([s, s, s], axis=2)
    inv_k = jnp.float32(1.0 / KNN)
    agg_s = jnp.sum(msg_s, axis=0) * inv_k
    agg_v = jnp.sum(msg_v, axis=0) * inv_k
    gate = jax.nn.sigmoid(agg_s @ Wg_ref[...] + bg_ref[...])
    lhs = jnp.concatenate(
        [agg_s * jax.nn.sigmoid(agg_s),
         gate * agg_v[:, 0:C],
         gate * agg_v[:, C:2 * C],
         gate * agg_v[:, 2 * C:3 * C]], axis=0)
    mm = lhs @ Wcat_ref[...]
    upd = jnp.concatenate(
        [mm[0:PB, 0:C], mm[PB:2 * PB, C:2 * C],
         mm[2 * PB:3 * PB, C:2 * C], mm[3 * PB:4 * PB, C:2 * C]], axis=1)
    out_ref[0] = tbl_ref[0] + upd


def _layer0_body(rbf_ref, rhb_ref, emb_ref,
                 W1_ref, b1_ref, W2_ref, b2_ref,
                 Wg_ref, bg_ref, Wcat_ref, out_ref, *, C):
    rb2 = rbf_ref[0].reshape(KNN * PB, rbf_ref.shape[-1])
    h = rb2 @ W1_ref[...] + b1_ref[...]
    h = h * jax.nn.sigmoid(h)
    w2d = h @ W2_ref[...] + b2_ref[...]
    w = w2d.reshape(KNN, PB, 8 * C)
    wss = w[..., 0:C]
    wsv3 = w[..., 2 * C:5 * C]
    RB = rhb_ref[0]
    emb = emb_ref[...]
    msg_s = wss * emb[0:1, None, 0:C]
    msg_v = (wsv3 * RB) * emb[0:1, None, C:4 * C]
    inv_k = jnp.float32(1.0 / KNN)
    agg_s = jnp.sum(msg_s, axis=0) * inv_k
    agg_v = jnp.sum(msg_v, axis=0) * inv_k
    gate = jax.nn.sigmoid(agg_s @ Wg_ref[...] + bg_ref[...])
    lhs = jnp.concatenate(
        [agg_s * jax.nn.sigmoid(agg_s),
         gate * agg_v[:, 0:C],
         gate * agg_v[:, C:2 * C],
         gate * agg_v[:, 2 * C:3 * C]], axis=0)
    mm = lhs @ Wcat_ref[...]
    upd = jnp.concatenate(
        [mm[0:PB, 0:C] + emb[0:1, 0:C], mm[PB:2 * PB, C:2 * C],
         mm[2 * PB:3 * PB, C:2 * C], mm[3 * PB:4 * PB, C:2 * C]], axis=1)
    out_ref[0] = upd


def _readout_body(tbl_ref, Wc1_ref, bc1_ref, Wc2_ref, bc2_ref,
                  Wc3_ref, bc3_ref, out_ref, *, C):
    s = tbl_ref[0][:, 0:C]
    pooled = jnp.mean(s, axis=0, keepdims=True)
    h = pooled @ Wc1_ref[...] + bc1_ref[...]
    h = h * jax.nn.sigmoid(h)
    h = h @ Wc2_ref[...] + bc2_ref[...]
    h = h * jax.nn.sigmoid(h)
    out_ref[0] = h @ Wc3_ref[...] + bc3_ref[...]


def _make_sc_gather(R, D, n_workers, chunk, tc_tiling=True):
    per_w = R // n_workers
    n_chunks = per_w // chunk
    mesh = plsc.VectorSubcoreMesh(core_axis_name="c", subcore_axis_name="s")
    info = plsc.get_sparse_core_info()
    nc = info.num_cores

    @functools.partial(
        pl.kernel, mesh=mesh,
        out_type=jax.ShapeDtypeStruct((R, D), jnp.float32),
        compiler_params=pltpu.CompilerParams(use_tc_tiling_on_sc=tc_tiling),
        scratch_types=[
            pltpu.VMEM((chunk,), jnp.int32),
            pltpu.VMEM((chunk,), jnp.int32),
            pltpu.VMEM((chunk, D), jnp.float32),
            pltpu.VMEM((chunk, D), jnp.float32),
            pltpu.SemaphoreType.DMA,
            pltpu.SemaphoreType.DMA,
            pltpu.SemaphoreType.DMA,
            pltpu.SemaphoreType.DMA,
        ],
    )
    def gather_k(table_hbm, idx_hbm, out_hbm, idx_v0, idx_v1,
                 rows_v0, rows_v1, g0, g1, s0, s1):
        assert n_chunks % 2 == 0 and n_chunks >= 2
        wid = lax.axis_index("s") * nc + lax.axis_index("c")
        idx_v = (idx_v0, idx_v1)
        rows_v = (rows_v0, rows_v1)
        gsem = (g0, g1)
        ssem = (s0, s1)

        def start(j, slot):
            base = pl.multiple_of(wid * per_w + j * chunk, 8)
            pltpu.sync_copy(idx_hbm.at[pl.ds(base, chunk)], idx_v[slot])
            pltpu.async_copy(table_hbm.at[idx_v[slot]], rows_v[slot],
                             gsem[slot])

        def wait_scatter(j, slot):
            base = pl.multiple_of(wid * per_w + j * chunk, 8)
            pltpu.make_async_copy(rows_v[slot], out_hbm.at[pl.ds(base, chunk)],
                                  ssem[slot]).wait()

        start(0, 0)

        def body(jj, carry):
            for b in range(2):
                j = 2 * jj + b
                nb = 1 - b

                @pl.when(j + 1 < n_chunks)
                def _():
                    @pl.when(j >= 1)
                    def _():
                        wait_scatter(j - 1, nb)
                    start(j + 1, nb)

                base = pl.multiple_of(wid * per_w + j * chunk, 8)
                pltpu.make_async_copy(table_hbm.at[idx_v[b]], rows_v[b],
                                      gsem[b]).wait()
                pltpu.async_copy(rows_v[b], out_hbm.at[pl.ds(base, chunk)],
                                 ssem[b])
            return carry

        lax.fori_loop(0, n_chunks // 2, body, 0)
        wait_scatter(n_chunks - 2, 0)
        wait_scatter(n_chunks - 1, 1)

    return gather_k


def kernel(batch, embed_w, W1, b1, W2, b2, Wg, bg, Wms, Wmv,
           Wc1, bc1, Wc2, bc2, Wc3, bc3):
    B, P, _ = batch.shape
    C = embed_w.shape[1]
    num_rbf = W1.shape[1]
    RH = W1.shape[2]
    L = W1.shape[0]
    ncls = Wc3.shape[1]
    nblk = P // PB

    batchT = jnp.transpose(batch, (0, 2, 1))

    nbr, rbf = pl.pallas_call(
        functools.partial(_knn_body, P=P, num_rbf=num_rbf),
        grid=(B, nblk),
        in_specs=[
            pl.BlockSpec((1, PB, 3), lambda b, i: (b, i, 0)),
            pl.BlockSpec((1, 3, P), lambda b, i: (b, 0, 0)),
        ],
        out_specs=[
            pl.BlockSpec((1, PB, KNN), lambda b, i: (b, i, 0)),
            pl.BlockSpec((1, KNN, PB, num_rbf), lambda b, i: (b, 0, i, 0)),
        ],
        out_shape=[
            jax.ShapeDtypeStruct((B, P, KNN), jnp.int32),
            jax.ShapeDtypeStruct((B, KNN, P, num_rbf), jnp.float32),
        ],
    )(batch, batchT)

    idx = jnp.transpose(nbr, (0, 2, 1)).reshape(B * P * KNN)
    R = B * P * KNN
    D = 4 * C
    sc_gather = _make_sc_gather(R, D, 32, 256)
    sc_gather_pos = _make_sc_gather(R, 16, 32, 512, tc_tiling=False)

    pos_table = jnp.pad(batch.reshape(B * P, 3), ((0, 0), (0, 13)))
    psrc = sc_gather_pos(pos_table, idx)

    rhb = pl.pallas_call(
        functools.partial(_edge_body, C=C),
        grid=(B, nblk),
        in_specs=[
            pl.BlockSpec((1, KNN, PB, 16), lambda b, i: (b, 0, i, 0)),
            pl.BlockSpec((1, PB, 3), lambda b, i: (b, i, 0)),
        ],
        out_specs=pl.BlockSpec((1, KNN, PB, 3 * C), lambda b, i: (b, 0, i, 0)),
        out_shape=jax.ShapeDtypeStruct((B, KNN, P, 3 * C), jnp.float32),
    )(psrc.reshape(B, KNN, P, 16), batch)

    emb4 = jnp.concatenate([embed_w, embed_w, embed_w, embed_w], axis=1)

    full = lambda shape: pl.BlockSpec(shape, lambda b, i: tuple(0 for _ in shape))
    def make_layer_call(nb):
        return pl.pallas_call(
            functools.partial(_layer_body, C=C),
            grid=(nb, nblk),
            in_specs=[
                pl.BlockSpec((1, KNN, PB, 4 * C), lambda b, i: (b, 0, i, 0)),
                pl.BlockSpec((1, KNN, PB, num_rbf), lambda b, i: (b, 0, i, 0)),
                pl.BlockSpec((1, KNN, PB, 3 * C), lambda b, i: (b, 0, i, 0)),
                pl.BlockSpec((1, PB, 4 * C), lambda b, i: (b, i, 0)),
                full((num_rbf, RH)), full((1, RH)),
                full((RH, 8 * C)), full((1, 8 * C)),
                full((C, C)), full((1, C)), full((C, 2 * C)),
            ],
            out_specs=pl.BlockSpec((1, PB, 4 * C), lambda b, i: (b, i, 0)),
            out_shape=jax.ShapeDtypeStruct((nb, P, 4 * C), jnp.float32),
        )

    layer_call = make_layer_call(B)

    layer0_call = pl.pallas_call(
        functools.partial(_layer0_body, C=C),
        grid=(B, nblk),
        in_specs=[
            pl.BlockSpec((1, KNN, PB, num_rbf), lambda b, i: (b, 0, i, 0)),
            pl.BlockSpec((1, KNN, PB, 3 * C), lambda b, i: (b, 0, i, 0)),
            full((1, 4 * C)),
            full((num_rbf, RH)), full((1, RH)),
            full((RH, 8 * C)), full((1, 8 * C)),
            full((C, C)), full((1, C)), full((C, 2 * C)),
        ],
        out_specs=pl.BlockSpec((1, PB, 4 * C), lambda b, i: (b, i, 0)),
        out_shape=jax.ShapeDtypeStruct((B, P, 4 * C), jnp.float32),
    )

    table = None
    for l in range(L):
        W2l, b2l = W2[l], b2[l]
        W2rep = jnp.concatenate(
            [W2l[:, 0:2 * C],
             W2l[:, 2 * C:3 * C], W2l[:, 2 * C:3 * C], W2l[:, 2 * C:3 * C],
             W2l[:, 3 * C:4 * C], W2l[:, 3 * C:4 * C], W2l[:, 3 * C:4 * C]],
            axis=1)
        b2rep = jnp.concatenate(
            [b2l[0:2 * C],
             b2l[2 * C:3 * C], b2l[2 * C:3 * C], b2l[2 * C:3 * C],
             b2l[3 * C:4 * C], b2l[3 * C:4 * C], b2l[3 * C:4 * C]])
        Wcat = jnp.concatenate([Wms[l], Wmv[l]], axis=1)
        if l == 0:
            table3 = layer0_call(
                rbf, rhb, emb4,
                W1[l], b1[l][None, :], W2rep, b2rep[None, :],
                Wg[l], bg[l][None, :], Wcat)
        else:
            g = sc_gather(table, idx)
            table3 = layer_call(
                g.reshape(B, KNN, P, 4 * C),
                rbf, rhb, table.reshape(B, P, 4 * C),
                W1[l], b1[l][None, :], W2rep, b2rep[None, :],
                Wg[l], bg[l][None, :], Wcat)
        table = table3.reshape(B * P, 4 * C)

    out = pl.pallas_call(
        functools.partial(_readout_body, C=C),
        grid=(B,),
        in_specs=[
            pl.BlockSpec((1, P, 4 * C), lambda b: (b, 0, 0)),
            pl.BlockSpec((C, 128), lambda b: (0, 0)),
            pl.BlockSpec((1, 128), lambda b: (0, 0)),
            pl.BlockSpec((128, 64), lambda b: (0, 0)),
            pl.BlockSpec((1, 64), lambda b: (0, 0)),
            pl.BlockSpec((64, ncls), lambda b: (0, 0)),
            pl.BlockSpec((1, ncls), lambda b: (0, 0)),
        ],
        out_specs=pl.BlockSpec((1, 1, ncls), lambda b: (b, 0, 0)),
        out_shape=jax.ShapeDtypeStruct((B, 1, ncls), jnp.float32),
    )(table.reshape(B, P, 4 * C), Wc1, bc1[None, :], Wc2, bc2[None, :],
      Wc3, bc3[None, :])

    return out.reshape(B, ncls)
